# Initial kernel scaffold; baseline (speedup 1.0000x reference)
import jax
import jax.numpy as jnp
from jax.experimental import pallas as pl


def kernel(x, weight):
    raise NotImplementedError("write your pallas kernel here")


# speedup vs baseline: 9.2296x; 9.2296x over previous
import jax
import jax.numpy as jnp
from jax import lax
from jax.experimental import pallas as pl
from jax.experimental.pallas import tpu as pltpu
from jax.experimental.pallas import tpu_sc as plsc

D_MODEL = 128
CHUNK = 128


def _embed_lookup(x_flat, weight, nw, n_chunks):
    mesh = plsc.VectorSubcoreMesh(core_axis_name="c", subcore_axis_name="s")
    nc = mesh.num_cores

    @pl.kernel(
        out_type=jax.ShapeDtypeStruct((nw, n_chunks, CHUNK, D_MODEL), jnp.float32),
        mesh=mesh,
        scratch_types=[
            pltpu.VMEM((n_chunks, CHUNK), jnp.int32),
            pltpu.VMEM((CHUNK, D_MODEL), jnp.float32),
            pltpu.VMEM((CHUNK, D_MODEL), jnp.float32),
            pltpu.SemaphoreType.DMA,
            pltpu.SemaphoreType.DMA,
        ],
    )
    def k(x_hbm, w_hbm, out_hbm, idx_v, buf0, buf1, sem0, sem1):
        wid = lax.axis_index("s") * nc + lax.axis_index("c")
        pltpu.sync_copy(x_hbm.at[wid], idx_v)

        bufs = (buf0, buf1)
        sems = (sem0, sem1)

        pltpu.async_copy(w_hbm.at[idx_v.at[0]], buf0, sem0)

        @pl.loop(0, n_chunks - 2, step=2)
        def _(j):
            for b in range(2):
                jj = j + b
                pltpu.async_copy(w_hbm.at[idx_v.at[jj + 1]], bufs[1 - b], sems[1 - b])
                pltpu.make_async_copy(w_hbm.at[idx_v.at[jj]], bufs[b], sems[b]).wait()
                pltpu.sync_copy(bufs[b], out_hbm.at[wid, jj])

        pltpu.async_copy(w_hbm.at[idx_v.at[n_chunks - 1]], buf1, sem1)
        pltpu.make_async_copy(w_hbm.at[idx_v.at[0]], buf0, sem0).wait()
        pltpu.sync_copy(buf0, out_hbm.at[wid, n_chunks - 2])
        pltpu.make_async_copy(w_hbm.at[idx_v.at[0]], buf1, sem1).wait()
        pltpu.sync_copy(buf1, out_hbm.at[wid, n_chunks - 1])

    return k(x_flat, weight)


@jax.jit
def kernel(x, weight):
    batch, hist = x.shape
    total = batch * hist
    info = plsc.get_sparse_core_info()
    nw = info.num_cores * info.num_subcores
    n_chunks = total // (nw * CHUNK)
    x_flat = x.astype(jnp.int32).reshape(nw, n_chunks, CHUNK)
    out = _embed_lookup(x_flat, weight, nw, n_chunks)
    return out.reshape(batch, hist, weight.shape[1])


# speedup vs baseline: 9.2939x; 1.0070x over previous
import jax
import jax.numpy as jnp
from jax import lax
from jax.experimental import pallas as pl
from jax.experimental.pallas import tpu as pltpu
from jax.experimental.pallas import tpu_sc as plsc

D_MODEL = 128
CHUNK = 128


def _embed_lookup(x_flat, weight, nw, n_chunks):
    mesh = plsc.VectorSubcoreMesh(core_axis_name="c", subcore_axis_name="s")
    nc = mesh.num_cores

    nbuf = 4
    assert n_chunks % nbuf == 0 and n_chunks > 2 * nbuf

    @pl.kernel(
        out_type=jax.ShapeDtypeStruct((nw, n_chunks, CHUNK, D_MODEL), jnp.float32),
        mesh=mesh,
        scratch_types=[
            pltpu.VMEM((n_chunks, CHUNK), jnp.int32),
            *([pltpu.VMEM((CHUNK, D_MODEL), jnp.float32)] * nbuf),
            *([pltpu.SemaphoreType.DMA] * (2 * nbuf)),
        ],
    )
    def k(x_hbm, w_hbm, out_hbm, idx_v, *bufs_and_sems):
        bufs = bufs_and_sems[:nbuf]
        gsem = bufs_and_sems[nbuf : 2 * nbuf]
        ssem = bufs_and_sems[2 * nbuf :]
        wid = lax.axis_index("s") * nc + lax.axis_index("c")
        pltpu.sync_copy(x_hbm.at[wid], idx_v)

        def wait_gather(b):
            pltpu.make_async_copy(w_hbm.at[idx_v.at[0]], bufs[b], gsem[b]).wait()

        def wait_store(b, t):
            pltpu.make_async_copy(bufs[b], out_hbm.at[wid, t], ssem[b]).wait()

        for b in range(nbuf):
            pltpu.async_copy(w_hbm.at[idx_v.at[b]], bufs[b], gsem[b])

        @pl.loop(0, n_chunks - nbuf, step=nbuf)
        def _(j):
            for b in range(nbuf):
                t = j + b
                wait_gather(b)
                pltpu.async_copy(bufs[b], out_hbm.at[wid, t], ssem[b])
                wait_store(b, t)
                pltpu.async_copy(w_hbm.at[idx_v.at[t + nbuf]], bufs[b], gsem[b])

        for b in range(nbuf):
            t = n_chunks - nbuf + b
            wait_gather(b)
            pltpu.async_copy(bufs[b], out_hbm.at[wid, t], ssem[b])
        for b in range(nbuf):
            wait_store(b, n_chunks - nbuf + b)

    return k(x_flat, weight)


@jax.jit
def kernel(x, weight):
    batch, hist = x.shape
    total = batch * hist
    info = plsc.get_sparse_core_info()
    nw = info.num_cores * info.num_subcores
    n_chunks = total // (nw * CHUNK)
    x_flat = x.astype(jnp.int32).reshape(nw, n_chunks, CHUNK)
    out = _embed_lookup(x_flat, weight, nw, n_chunks)
    return out.reshape(batch, hist, weight.shape[1])


# speedup vs baseline: 9.2962x; 1.0002x over previous
import jax
import jax.numpy as jnp
from jax import lax
from jax.experimental import pallas as pl
from jax.experimental.pallas import tpu as pltpu
from jax.experimental.pallas import tpu_sc as plsc

D_MODEL = 128
CHUNK = 128


def _embed_lookup(x_flat, weight, nw, n_chunks):
    mesh = plsc.VectorSubcoreMesh(core_axis_name="c", subcore_axis_name="s")
    nc = mesh.num_cores

    nbuf = 5
    assert n_chunks % nbuf == 0 and n_chunks > 2 * nbuf

    @pl.kernel(
        out_type=jax.ShapeDtypeStruct((nw, n_chunks, CHUNK, D_MODEL), jnp.float32),
        mesh=mesh,
        scratch_types=[
            pltpu.VMEM((n_chunks, CHUNK), jnp.int32),
            *([pltpu.VMEM((CHUNK, D_MODEL), jnp.float32)] * nbuf),
            *([pltpu.SemaphoreType.DMA] * (2 * nbuf)),
        ],
    )
    def k(x_hbm, w_hbm, out_hbm, idx_v, *bufs_and_sems):
        bufs = bufs_and_sems[:nbuf]
        gsem = bufs_and_sems[nbuf : 2 * nbuf]
        ssem = bufs_and_sems[2 * nbuf :]
        wid = lax.axis_index("s") * nc + lax.axis_index("c")
        pltpu.sync_copy(x_hbm.at[wid], idx_v)

        def wait_gather(b):
            pltpu.make_async_copy(w_hbm.at[idx_v.at[0]], bufs[b], gsem[b]).wait()

        def wait_store(b, t):
            pltpu.make_async_copy(bufs[b], out_hbm.at[wid, t], ssem[b]).wait()

        for b in range(nbuf):
            pltpu.async_copy(w_hbm.at[idx_v.at[b]], bufs[b], gsem[b])

        @pl.loop(0, n_chunks - nbuf, step=nbuf)
        def _(j):
            for b in range(nbuf):
                t = j + b
                wait_gather(b)
                pltpu.async_copy(bufs[b], out_hbm.at[wid, t], ssem[b])
                wait_store(b, t)
                pltpu.async_copy(w_hbm.at[idx_v.at[t + nbuf]], bufs[b], gsem[b])

        for b in range(nbuf):
            t = n_chunks - nbuf + b
            wait_gather(b)
            pltpu.async_copy(bufs[b], out_hbm.at[wid, t], ssem[b])
        for b in range(nbuf):
            wait_store(b, n_chunks - nbuf + b)

    return k(x_flat, weight)


@jax.jit
def kernel(x, weight):
    batch, hist = x.shape
    total = batch * hist
    info = plsc.get_sparse_core_info()
    nw = info.num_cores * info.num_subcores
    n_chunks = total // (nw * CHUNK)
    x_flat = x.astype(jnp.int32).reshape(nw, n_chunks, CHUNK)
    out = _embed_lookup(x_flat, weight, nw, n_chunks)
    return out.reshape(batch, hist, weight.shape[1])
